# baseline (speedup 1.0000x reference)
import math

import jax
import jax.numpy as jnp
from jax import lax
from jax.experimental import pallas as pl
from jax.experimental.pallas import tpu as pltpu
from jax.experimental.pallas import tpu_sc as plsc

D_MODEL = 64
VOCAB = 1000000
BATCH = 4096
HIST = 200
SCALE = math.sqrt(D_MODEL)

NC = 2
NS = 16
NW = NC * NS

B_TOT = BATCH * HIST
B_PER_W = B_TOT // NW
G = 64
N_GROUPS = B_PER_W // G
NBUF = 4


N_TILES = VOCAB // 8
TCHUNK = 24
DBUF = 2


DSTEPS = (1954 + TCHUNK // 2 - 1) // (TCHUNK // 2)


def _depad_body(lut4_hbm, pairs_hbm, slabs, stages, rsem, wsem):
    cid = lax.axis_index("c")
    sid = lax.axis_index("s")
    wid = sid * NC + cid
    lo = wid * 1953 + jnp.minimum(wid, 4)
    hi = lo + 1953 + jnp.where(wid < 4, 1, 0)

    def tstart(s):
        d = jnp.minimum(lo + s * (TCHUNK // 2), hi - TCHUNK // 2)
        return pl.multiple_of(d * 2, 2)

    @pl.loop(0, DSTEPS, step=DBUF)
    def _(s0):
        for b in range(DBUF):
            pltpu.async_copy(
                lut4_hbm.at[pl.ds(tstart(s0 + b), TCHUNK)], slabs.at[b], rsem.at[b]
            )
        for b in range(DBUF):
            pltpu.make_async_copy(
                lut4_hbm.at[pl.ds(tstart(s0 + b), TCHUNK)], slabs.at[b], rsem.at[b]
            ).wait()
            for t in range(TCHUNK):
                for k in range(4):
                    for j in range(D_MODEL // 16):
                        stages[b, t * 4 + k, pl.ds(j * 16, 16)] = (
                            slabs[b, t, 2 * k, pl.ds(j * 16, 16)]
                        )
                        stages[b, t * 4 + k, pl.ds(D_MODEL + j * 16, 16)] = (
                            slabs[b, t, 2 * k + 1, pl.ds(j * 16, 16)]
                        )
            pltpu.async_copy(
                stages.at[b],
                pairs_hbm.at[pl.ds(tstart(s0 + b) * 4, TCHUNK * 4)],
                wsem.at[b],
            )
        for b in range(DBUF):
            pltpu.make_async_copy(
                stages.at[b],
                pairs_hbm.at[pl.ds(tstart(s0 + b) * 4, TCHUNK * 4)],
                wsem.at[b],
            ).wait()


def _body(idx_hbm, pairs_hbm, out_hbm, idx_v, idx_p, bufs, obufs, gsem, osem):
    cid = lax.axis_index("c")
    sid = lax.axis_index("s")
    wid = sid * NC + cid
    base = wid * B_PER_W

    pltpu.sync_copy(idx_hbm.at[wid], idx_v)

    eight = jnp.full((16,), SCALE, jnp.float32)

    @pl.loop(0, N_GROUPS, step=NBUF)
    def _(g0):
        for b in range(NBUF):
            for j in range(G // 16):
                v = idx_v[g0 + b, pl.ds(j * 16, 16)]
                idx_p[b, pl.ds(j * 16, 16)] = jax.lax.shift_right_logical(v, 1)
            pltpu.async_copy(
                pairs_hbm.at[idx_p.at[b]], bufs.at[b], gsem.at[b]
            )
        for b in range(NBUF):
            pltpu.make_async_copy(
                pairs_hbm.at[idx_p.at[b]], bufs.at[b], gsem.at[b]
            ).wait()

            @pl.loop(0, G // 16)
            def _(rg):
                vvec = idx_v[g0 + b, pl.ds(rg * 16, 16)]
                pvec = jax.lax.bitwise_and(vvec, 1)
                for k in range(16):
                    i = rg * 16 + k
                    pk = jax.lax.gather(
                        pvec,
                        jnp.full((16, 1), k, jnp.int32),
                        jax.lax.GatherDimensionNumbers(
                            offset_dims=(),
                            collapsed_slice_dims=(0,),
                            start_index_map=(0,),
                        ),
                        (1,),
                        mode=jax.lax.GatherScatterMode.PROMISE_IN_BOUNDS,
                    )
                    odd = pk == 1
                    for j in range(D_MODEL // 16):
                        lo = bufs[b, i, pl.ds(j * 16, 16)]
                        hi = bufs[b, i, pl.ds(D_MODEL + j * 16, 16)]
                        obufs[b, i, pl.ds(j * 16, 16)] = (
                            jnp.where(odd, hi, lo) * eight
                        )

            pltpu.async_copy(
                obufs.at[b], out_hbm.at[pl.ds(base + (g0 + b) * G, G)], osem.at[b]
            )
        for b in range(NBUF):
            pltpu.make_async_copy(
                obufs.at[b], out_hbm.at[pl.ds(base, G)], osem.at[b]
            ).wait()


@jax.jit
def _run(idx, lut4):
    mesh = plsc.VectorSubcoreMesh(core_axis_name="c", subcore_axis_name="s")
    depad = pl.kernel(
        _depad_body,
        out_type=jax.ShapeDtypeStruct((VOCAB // 2, 2 * D_MODEL), jnp.float32),
        mesh=mesh,
        scratch_types=[
            pltpu.VMEM((DBUF, TCHUNK, 8, D_MODEL), jnp.float32),
            pltpu.VMEM((DBUF, TCHUNK * 4, 2 * D_MODEL), jnp.float32),
            pltpu.SemaphoreType.DMA((DBUF,)),
            pltpu.SemaphoreType.DMA((DBUF,)),
        ],
        compiler_params=pltpu.CompilerParams(
            use_tc_tiling_on_sc=True, needs_layout_passes=False
        ),
    )
    pairs = depad(lut4)
    f = pl.kernel(
        _body,
        out_type=jax.ShapeDtypeStruct((B_TOT, D_MODEL), jnp.float32),
        mesh=mesh,
        scratch_types=[
            pltpu.VMEM((N_GROUPS, G), jnp.int32),
            pltpu.VMEM((NBUF, G), jnp.int32),
            pltpu.VMEM((NBUF, G, 2 * D_MODEL), jnp.float32),
            pltpu.VMEM((NBUF, G, D_MODEL), jnp.float32),
            pltpu.SemaphoreType.DMA((NBUF,)),
            pltpu.SemaphoreType.DMA((NBUF,)),
        ],
        compiler_params=pltpu.CompilerParams(
            use_tc_tiling_on_sc=True, needs_layout_passes=False
        ),
    )
    return f(idx, pairs)


def kernel(x, lut):
    idx = x.astype(jnp.int32).reshape(NW, N_GROUPS, G)
    lut4 = lut.reshape(N_TILES, 8, D_MODEL)
    out = _run(idx, lut4)
    return out.reshape(BATCH, HIST, D_MODEL)


# speedup vs baseline: 1.0715x; 1.0715x over previous
import math

import jax
import jax.numpy as jnp
from jax import lax
from jax.experimental import pallas as pl
from jax.experimental.pallas import tpu as pltpu
from jax.experimental.pallas import tpu_sc as plsc

D_MODEL = 64
VOCAB = 1000000
BATCH = 4096
HIST = 200
SCALE = math.sqrt(D_MODEL)

NC = 2
NS = 16
NW = NC * NS

B_TOT = BATCH * HIST
B_PER_W = B_TOT // NW
G = 128
N_GROUPS = B_PER_W // G
NBUF = 2


N_TILES = VOCAB // 8
TCHUNK = 16
DBUF = 2


DSTEPS = (1954 + TCHUNK // 2 - 1) // (TCHUNK // 2)


def _depad_body(lut4_hbm, pairs_hbm, slabs, stages, rsem, wsem):
    cid = lax.axis_index("c")
    sid = lax.axis_index("s")
    wid = sid * NC + cid
    lo = wid * 1953 + jnp.minimum(wid, 4)
    hi = lo + 1953 + jnp.where(wid < 4, 1, 0)

    def tstart(s):
        d = jnp.minimum(lo + s * (TCHUNK // 2), hi - TCHUNK // 2)
        return pl.multiple_of(d * 2, 2)

    @pl.loop(0, DSTEPS, step=DBUF)
    def _(s0):
        for b in range(DBUF):
            pltpu.async_copy(
                lut4_hbm.at[pl.ds(tstart(s0 + b), TCHUNK)], slabs.at[b], rsem.at[b]
            )
        for b in range(DBUF):
            pltpu.make_async_copy(
                lut4_hbm.at[pl.ds(tstart(s0 + b), TCHUNK)], slabs.at[b], rsem.at[b]
            ).wait()
            for t in range(TCHUNK):
                for k in range(4):
                    for j in range(D_MODEL // 16):
                        stages[b, t * 4 + k, pl.ds(j * 16, 16)] = (
                            slabs[b, t, 2 * k, pl.ds(j * 16, 16)]
                        )
                        stages[b, t * 4 + k, pl.ds(D_MODEL + j * 16, 16)] = (
                            slabs[b, t, 2 * k + 1, pl.ds(j * 16, 16)]
                        )
            pltpu.async_copy(
                stages.at[b],
                pairs_hbm.at[pl.ds(tstart(s0 + b) * 4, TCHUNK * 4)],
                wsem.at[b],
            )
        for b in range(DBUF):
            pltpu.make_async_copy(
                stages.at[b],
                pairs_hbm.at[pl.ds(tstart(s0 + b) * 4, TCHUNK * 4)],
                wsem.at[b],
            ).wait()


def _body(idx_hbm, pairs_hbm, out_hbm, idx_v, idx_p, bufs, obufs, gsem, osem):
    cid = lax.axis_index("c")
    sid = lax.axis_index("s")
    wid = sid * NC + cid
    base = wid * B_PER_W

    pltpu.sync_copy(idx_hbm.at[wid], idx_v)

    @pl.loop(0, N_GROUPS)
    def _(g):
        for j in range(G // 16):
            v = idx_v[g, pl.ds(j * 16, 16)]
            idx_p[g, pl.ds(j * 16, 16)] = jax.lax.shift_right_logical(v, 1)

    eight = jnp.full((16,), SCALE, jnp.float32)

    @pl.loop(0, N_GROUPS, step=NBUF)
    def _(g0):
        for b in range(NBUF):
            pltpu.async_copy(
                pairs_hbm.at[idx_p.at[g0 + b]], bufs.at[b], gsem.at[b]
            )
        for b in range(NBUF):
            pltpu.make_async_copy(
                pairs_hbm.at[idx_p.at[g0 + b]], bufs.at[b], gsem.at[b]
            ).wait()

            @pl.loop(0, G // 16)
            def _(rg):
                vvec = idx_v[g0 + b, pl.ds(rg * 16, 16)]
                pvec = jax.lax.bitwise_and(vvec, 1)
                for k in range(16):
                    i = rg * 16 + k
                    pk = jax.lax.gather(
                        pvec,
                        jnp.full((16, 1), k, jnp.int32),
                        jax.lax.GatherDimensionNumbers(
                            offset_dims=(),
                            collapsed_slice_dims=(0,),
                            start_index_map=(0,),
                        ),
                        (1,),
                        mode=jax.lax.GatherScatterMode.PROMISE_IN_BOUNDS,
                    )
                    odd = pk == 1
                    for j in range(D_MODEL // 16):
                        lo = bufs[b, i, pl.ds(j * 16, 16)]
                        hi = bufs[b, i, pl.ds(D_MODEL + j * 16, 16)]
                        obufs[b, i, pl.ds(j * 16, 16)] = (
                            jnp.where(odd, hi, lo) * eight
                        )

            pltpu.async_copy(
                obufs.at[b], out_hbm.at[pl.ds(base + (g0 + b) * G, G)], osem.at[b]
            )
        for b in range(NBUF):
            pltpu.make_async_copy(
                obufs.at[b], out_hbm.at[pl.ds(base, G)], osem.at[b]
            ).wait()


@jax.jit
def _run(idx, lut4):
    mesh = plsc.VectorSubcoreMesh(core_axis_name="c", subcore_axis_name="s")
    depad = pl.kernel(
        _depad_body,
        out_type=jax.ShapeDtypeStruct((VOCAB // 2, 2 * D_MODEL), jnp.float32),
        mesh=mesh,
        scratch_types=[
            pltpu.VMEM((DBUF, TCHUNK, 8, D_MODEL), jnp.float32),
            pltpu.VMEM((DBUF, TCHUNK * 4, 2 * D_MODEL), jnp.float32),
            pltpu.SemaphoreType.DMA((DBUF,)),
            pltpu.SemaphoreType.DMA((DBUF,)),
        ],
        compiler_params=pltpu.CompilerParams(
            use_tc_tiling_on_sc=True, needs_layout_passes=False
        ),
    )
    pairs = depad(lut4)
    f = pl.kernel(
        _body,
        out_type=jax.ShapeDtypeStruct((B_TOT, D_MODEL), jnp.float32),
        mesh=mesh,
        scratch_types=[
            pltpu.VMEM((N_GROUPS, G), jnp.int32),
            pltpu.VMEM((N_GROUPS, G), jnp.int32),
            pltpu.VMEM((NBUF, G, 2 * D_MODEL), jnp.float32),
            pltpu.VMEM((NBUF, G, D_MODEL), jnp.float32),
            pltpu.SemaphoreType.DMA((NBUF,)),
            pltpu.SemaphoreType.DMA((NBUF,)),
        ],
        compiler_params=pltpu.CompilerParams(
            use_tc_tiling_on_sc=True, needs_layout_passes=False
        ),
    )
    return f(idx, pairs)


def kernel(x, lut):
    idx = x.astype(jnp.int32).reshape(NW, N_GROUPS, G)
    lut4 = lut.reshape(N_TILES, 8, D_MODEL)
    out = _run(idx, lut4)
    return out.reshape(BATCH, HIST, D_MODEL)
